# initial kernel scaffold (unmeasured)
import jax
import jax.numpy as jnp
from jax import lax
from jax.experimental import pallas as pl
from jax.experimental.pallas import tpu as pltpu


def kernel(
    x,
):
    def body(*refs):
        pass

    out_shape = jax.ShapeDtypeStruct(..., jnp.float32)
    return pl.pallas_call(body, out_shape=out_shape)(...)


# baseline (device time: 15853 ns/iter reference)
import jax
import jax.numpy as jnp
from jax import lax
from jax.experimental import pallas as pl
from jax.experimental.pallas import tpu as pltpu

N_DEV = 16


def kernel(x):
    m_per, n = x.shape

    def body(x_ref, out_ref, mine_ref, comm_ref, send_sem, recv_sem):
        my = lax.axis_index("i")

        xv = x_ref[...].astype(jnp.float32)
        row = lax.broadcasted_iota(jnp.int32, (m_per, m_per), 0)
        col = lax.broadcasted_iota(jnp.int32, (m_per, m_per), 1)
        tri = (row >= col).astype(jnp.float32)
        cs = lax.dot_general(
            tri, xv, (((1,), (0,)), ((), ())),
            preferred_element_type=jnp.float32,
        )

        mine_ref[...] = cs[m_per - 1:m_per, :]

        for j in range(N_DEV):
            @pl.when(j != my)
            def _():
                rdma = pltpu.make_async_remote_copy(
                    src_ref=mine_ref,
                    dst_ref=comm_ref.at[my],
                    send_sem=send_sem,
                    recv_sem=recv_sem,
                    device_id=(j,),
                    device_id_type=pl.DeviceIdType.MESH,
                )
                rdma.start()

        waiter = pltpu.make_async_remote_copy(
            src_ref=mine_ref,
            dst_ref=comm_ref.at[0],
            send_sem=send_sem,
            recv_sem=recv_sem,
            device_id=(0,),
            device_id_type=pl.DeviceIdType.MESH,
        )
        for _ in range(N_DEV - 1):
            waiter.wait_recv()

        slot = lax.broadcasted_iota(jnp.int32, (N_DEV, 1, n), 0)
        offset = jnp.sum(
            jnp.where(slot < my, comm_ref[...], 0.0), axis=0
        )
        out_ref[...] = cs + offset

        for _ in range(N_DEV - 1):
            waiter.wait_send()

    return pl.pallas_call(
        body,
        out_shape=jax.ShapeDtypeStruct((m_per, n), jnp.float32),
        in_specs=[pl.BlockSpec(memory_space=pltpu.VMEM)],
        out_specs=pl.BlockSpec(memory_space=pltpu.VMEM),
        scratch_shapes=[
            pltpu.VMEM((1, n), jnp.float32),
            pltpu.VMEM((N_DEV, 1, n), jnp.float32),
            pltpu.SemaphoreType.DMA,
            pltpu.SemaphoreType.DMA,
        ],
    )(x)


# device time: 9465 ns/iter; 1.6749x vs baseline; 1.6749x over previous
import jax
import jax.numpy as jnp
from jax import lax
from jax.experimental import pallas as pl
from jax.experimental.pallas import tpu as pltpu

N_DEV = 16


def kernel(x):
    m_per, n = x.shape

    def body(x_ref, out_ref, mine_ref, comm_ref, send_sem, recv_sem):
        my = lax.axis_index("i")

        barrier = pltpu.get_barrier_semaphore()
        for j in range(N_DEV):
            @pl.when(j != my)
            def _():
                pl.semaphore_signal(
                    barrier, inc=1,
                    device_id=(j,),
                    device_id_type=pl.DeviceIdType.MESH,
                )

        xv = x_ref[...].astype(jnp.float32)
        row = lax.broadcasted_iota(jnp.int32, (m_per, m_per), 0)
        col = lax.broadcasted_iota(jnp.int32, (m_per, m_per), 1)
        tri = (row >= col).astype(jnp.float32)
        cs = lax.dot_general(
            tri, xv, (((1,), (0,)), ((), ())),
            preferred_element_type=jnp.float32,
        )

        mine_ref[...] = cs[m_per - 1:m_per, :]

        pl.semaphore_wait(barrier, N_DEV - 1)

        for j in range(N_DEV):
            @pl.when(j != my)
            def _():
                rdma = pltpu.make_async_remote_copy(
                    src_ref=mine_ref,
                    dst_ref=comm_ref.at[my],
                    send_sem=send_sem,
                    recv_sem=recv_sem,
                    device_id=(j,),
                    device_id_type=pl.DeviceIdType.MESH,
                )
                rdma.start()

        out_ref[...] = cs

        waiter = pltpu.make_async_remote_copy(
            src_ref=mine_ref,
            dst_ref=comm_ref.at[0],
            send_sem=send_sem,
            recv_sem=recv_sem,
            device_id=(0,),
            device_id_type=pl.DeviceIdType.MESH,
        )
        for _ in range(N_DEV - 1):
            waiter.wait_recv()

        slot = lax.broadcasted_iota(jnp.int32, (N_DEV, 1, n), 0)
        offset = jnp.sum(
            jnp.where(slot < my, comm_ref[...], 0.0), axis=0
        )
        out_ref[...] = out_ref[...] + offset

        for _ in range(N_DEV - 1):
            waiter.wait_send()

    return pl.pallas_call(
        body,
        out_shape=jax.ShapeDtypeStruct((m_per, n), jnp.float32),
        in_specs=[pl.BlockSpec(memory_space=pltpu.VMEM)],
        out_specs=pl.BlockSpec(memory_space=pltpu.VMEM),
        scratch_shapes=[
            pltpu.VMEM((1, n), jnp.float32),
            pltpu.VMEM((N_DEV, 1, n), jnp.float32),
            pltpu.SemaphoreType.DMA,
            pltpu.SemaphoreType.DMA,
        ],
        compiler_params=pltpu.CompilerParams(collective_id=0),
    )(x)
